# baseline (speedup 1.0000x reference)
import numpy as np
import jax
import jax.numpy as jnp
from jax.experimental import pallas as pl
from jax.experimental.pallas import tpu as pltpu

_SLOW_FRAMES = 8
_CHUNK_SIZES = (4, 4, 4, 4, 4, 4, 4, 4)
_NBUF = 8
_LOOKAHEAD = 8


def _make_body(idx, T):
    slot_of = {t: j for j, t in enumerate(idx)}
    starts = np.concatenate([[0], np.cumsum(_CHUNK_SIZES)])
    assert starts[-1] == T
    chunks = [(int(starts[i]), int(w)) for i, w in enumerate(_CHUNK_SIZES)]
    n = len(chunks)

    def _body(frames_ref, slow_ref, fast_ref, buf, rsem, fsem, ssem):
        reads, fwrites, swrites = {}, {}, {}
        for k, (t0, width) in enumerate(chunks):
            b = k % _NBUF
            reads[k] = pltpu.make_async_copy(
                frames_ref.at[:, t0:t0 + width], buf.at[b, :, 0:width], rsem.at[b]
            )
            fwrites[k] = pltpu.make_async_copy(
                buf.at[b, :, 0:width], fast_ref.at[:, t0:t0 + width], fsem.at[b]
            )
            sw = []
            for o in range(width):
                t = t0 + o
                if t in slot_of:
                    j = slot_of[t]
                    sw.append(
                        pltpu.make_async_copy(
                            buf.at[b, :, o:o + 1],
                            slow_ref.at[:, j:j + 1],
                            ssem.at[b],
                        )
                    )
            if sw:
                swrites[k] = sw

        for step in range(n + _LOOKAHEAD):
            k = step
            if k < n:
                if k >= _NBUF:
                    fwrites[k - _NBUF].wait()
                    for c in swrites.get(k - _NBUF, ()):
                        c.wait()
                reads[k].start()
            u = step - _LOOKAHEAD
            if u >= 0:
                reads[u].wait()
                fwrites[u].start()
                for c in swrites.get(u, ()):
                    c.start()
        for k in range(max(0, n - _NBUF), n):
            fwrites[k].wait()
            for c in swrites.get(k, ()):
                c.wait()

    return _body


def kernel(frames):
    C, T, H, W = frames.shape
    idx = [int(v) for v in np.linspace(0.0, float(T - 1), _SLOW_FRAMES).astype(np.int32)]

    slow, fast = pl.pallas_call(
        _make_body(idx, T),
        in_specs=[pl.BlockSpec(memory_space=pltpu.MemorySpace.HBM)],
        out_specs=(
            pl.BlockSpec(memory_space=pltpu.MemorySpace.HBM),
            pl.BlockSpec(memory_space=pltpu.MemorySpace.HBM),
        ),
        out_shape=(
            jax.ShapeDtypeStruct((C, _SLOW_FRAMES, H, W), frames.dtype),
            jax.ShapeDtypeStruct((C, T, H, W), frames.dtype),
        ),
        scratch_shapes=[
            pltpu.VMEM((_NBUF, C, max(_CHUNK_SIZES), H, W), frames.dtype),
            pltpu.SemaphoreType.DMA((_NBUF,)),
            pltpu.SemaphoreType.DMA((_NBUF,)),
            pltpu.SemaphoreType.DMA((_NBUF,)),
        ],
    )(frames)
    return (slow, fast)


# speedup vs baseline: 1.0185x; 1.0185x over previous
import numpy as np
import jax
import jax.numpy as jnp
from jax.experimental import pallas as pl
from jax.experimental.pallas import tpu as pltpu

_SLOW_FRAMES = 8
_CHUNK_SIZES = (8, 8, 8, 8)
_NBUF = 4
_LOOKAHEAD = 4


def _make_body(idx, T):
    slot_of = {t: j for j, t in enumerate(idx)}
    starts = np.concatenate([[0], np.cumsum(_CHUNK_SIZES)])
    assert starts[-1] == T
    chunks = [(int(starts[i]), int(w)) for i, w in enumerate(_CHUNK_SIZES)]
    n = len(chunks)

    def _body(frames_ref, slow_ref, fast_ref, buf, rsem, fsem, ssem):
        reads, fwrites, swrites = {}, {}, {}
        for k, (t0, width) in enumerate(chunks):
            b = k % _NBUF
            reads[k] = pltpu.make_async_copy(
                frames_ref.at[:, t0:t0 + width], buf.at[b, :, 0:width], rsem.at[b]
            )
            fwrites[k] = pltpu.make_async_copy(
                buf.at[b, :, 0:width], fast_ref.at[:, t0:t0 + width], fsem.at[b]
            )
            sw = []
            for o in range(width):
                t = t0 + o
                if t in slot_of:
                    j = slot_of[t]
                    sw.append(
                        pltpu.make_async_copy(
                            buf.at[b, :, o:o + 1],
                            slow_ref.at[:, j:j + 1],
                            ssem.at[b],
                        )
                    )
            if sw:
                swrites[k] = sw

        for step in range(n + _LOOKAHEAD):
            k = step
            if k < n:
                if k >= _NBUF:
                    fwrites[k - _NBUF].wait()
                    for c in swrites.get(k - _NBUF, ()):
                        c.wait()
                reads[k].start()
            u = step - _LOOKAHEAD
            if u >= 0:
                reads[u].wait()
                fwrites[u].start()
                for c in swrites.get(u, ()):
                    c.start()
        for k in range(max(0, n - _NBUF), n):
            fwrites[k].wait()
            for c in swrites.get(k, ()):
                c.wait()

    return _body


def kernel(frames):
    C, T, H, W = frames.shape
    idx = [int(v) for v in np.linspace(0.0, float(T - 1), _SLOW_FRAMES).astype(np.int32)]

    slow, fast = pl.pallas_call(
        _make_body(idx, T),
        in_specs=[pl.BlockSpec(memory_space=pltpu.MemorySpace.HBM)],
        out_specs=(
            pl.BlockSpec(memory_space=pltpu.MemorySpace.HBM),
            pl.BlockSpec(memory_space=pltpu.MemorySpace.HBM),
        ),
        out_shape=(
            jax.ShapeDtypeStruct((C, _SLOW_FRAMES, H, W), frames.dtype),
            jax.ShapeDtypeStruct((C, T, H, W), frames.dtype),
        ),
        scratch_shapes=[
            pltpu.VMEM((_NBUF, C, max(_CHUNK_SIZES), H, W), frames.dtype),
            pltpu.SemaphoreType.DMA((_NBUF,)),
            pltpu.SemaphoreType.DMA((_NBUF,)),
            pltpu.SemaphoreType.DMA((_NBUF,)),
        ],
    )(frames)
    return (slow, fast)
